# Initial kernel scaffold; baseline (speedup 1.0000x reference)
import jax
import jax.numpy as jnp
from jax.experimental import pallas as pl


def kernel(pred_coeffs, observed, G, ref, obs_subsample):
    raise NotImplementedError("write your pallas kernel here")


# speedup vs baseline: 2.3797x; 2.3797x over previous
import functools

import jax
import jax.numpy as jnp
from jax.experimental import pallas as pl

FOCAL_UM = 6000.0
PITCH_UM = 150.0
SENSOR_W = 9600.0
SENSOR_H = 9600.0
GRID = 64
N_SUB = GRID * GRID
MARGIN = PITCH_UM * 0.5


def _chamfer_kernel(pred_ref, obs_ref, g_ref, ref_ref, valid_ref, out_ref):
    b = pl.program_id(0)

    coeffs = pred_ref[b, :].reshape(1, -1)
    g_tail = g_ref[:, 1:]
    slopes = jax.lax.dot_general(
        g_tail, coeffs, (((1,), (1,)), ((), ())),
        preferred_element_type=jnp.float32)

    ex = ref_ref[:, 0:1] + FOCAL_UM * slopes[:N_SUB]
    ey = ref_ref[:, 1:2] + FOCAL_UM * slopes[N_SUB:]

    in_bounds = ((ex >= -MARGIN) & (ex <= SENSOR_W + MARGIN)
                 & (ey >= -MARGIN) & (ey <= SENSOR_H + MARGIN))
    n_ib = jnp.sum(in_bounds.astype(jnp.float32))

    cx = SENSOR_W / 2.0
    cy = SENSOR_H / 2.0
    center_d = jnp.sqrt((ex - cx) ** 2 + (ey - cy) ** 2)
    fallback = jnp.mean(center_d) / PITCH_UM + 10.0

    ox = obs_ref[b, :, 0].reshape(1, -1)
    oy = obs_ref[b, :, 1].reshape(1, -1)

    dx = ex - ox
    dy = ey - oy
    sq = dx * dx + dy * dy
    masked = jnp.where(in_bounds, sq, jnp.inf)
    min_sq = jnp.min(masked, axis=0, keepdims=True)
    min_d = jnp.sqrt(min_sq + 1e-12) / PITCH_UM
    clamped = jnp.minimum(min_d, 5.0)

    valid = valid_ref[0:1, :]
    chamfer = jnp.sum(clamped * valid) / jnp.sum(valid)

    loss_b = jnp.where(n_ib < 5.0, fallback, chamfer)
    out_ref[pl.ds(b, 1), :] = loss_b.reshape(1, 1)


@jax.jit
def _run(pred_coeffs, observed, G, ref, valid):
    Bn = pred_coeffs.shape[0]
    losses = pl.pallas_call(
        _chamfer_kernel,
        grid=(Bn,),
        in_specs=[
            pl.BlockSpec(pred_coeffs.shape, lambda b: (0, 0)),
            pl.BlockSpec(observed.shape, lambda b: (0, 0, 0)),
            pl.BlockSpec(G.shape, lambda b: (0, 0)),
            pl.BlockSpec(ref.shape, lambda b: (0, 0)),
            pl.BlockSpec(valid.shape, lambda b: (0, 0)),
        ],
        out_specs=pl.BlockSpec((Bn, 1), lambda b: (0, 0)),
        out_shape=jax.ShapeDtypeStruct((Bn, 1), jnp.float32),
    )(pred_coeffs, observed, G, ref, valid)
    return jnp.mean(losses)


def kernel(pred_coeffs, observed, G, ref, obs_subsample):
    M = observed.shape[1]
    valid = (jnp.arange(M) < obs_subsample).astype(jnp.float32).reshape(1, M)
    return _run(pred_coeffs, observed, G, ref, valid)


# speedup vs baseline: 7.0492x; 2.9622x over previous
import functools

import jax
import jax.numpy as jnp
from jax.experimental import pallas as pl

FOCAL_UM = 6000.0
PITCH_UM = 150.0
SENSOR_W = 9600.0
SENSOR_H = 9600.0
GRID = 64
N_SUB = GRID * GRID
MARGIN = PITCH_UM * 0.5
FAR = 1e9


def _chamfer_kernel(pred_ref, obs_ref, gt_ref, reft_ref, valid_ref, out_ref):
    b = pl.program_id(0)

    coeffs = pred_ref[pl.ds(b, 1), :]
    slopes = jnp.dot(coeffs, gt_ref[...],
                     preferred_element_type=jnp.float32)

    ex = reft_ref[0:1, :] + FOCAL_UM * slopes[:, :N_SUB]
    ey = reft_ref[1:2, :] + FOCAL_UM * slopes[:, N_SUB:]

    in_bounds = ((ex >= -MARGIN) & (ex <= SENSOR_W + MARGIN)
                 & (ey >= -MARGIN) & (ey <= SENSOR_H + MARGIN))
    n_ib = jnp.sum(in_bounds.astype(jnp.float32))

    cx = SENSOR_W / 2.0
    cy = SENSOR_H / 2.0
    center_d = jnp.sqrt((ex - cx) ** 2 + (ey - cy) ** 2)
    fallback = jnp.sum(center_d) * (1.0 / (N_SUB * PITCH_UM)) + 10.0

    ex_eff = jnp.where(in_bounds, ex, FAR)

    obs = obs_ref[b]
    ox = obs[:, 0:1]
    oy = obs[:, 1:2]

    dx = ex_eff - ox
    dy = ey - oy
    sq = dx * dx + dy * dy
    min_sq = jnp.min(sq, axis=1, keepdims=True)
    min_d = jnp.sqrt(min_sq + 1e-12) * (1.0 / PITCH_UM)
    clamped = jnp.minimum(min_d, 5.0)

    valid = valid_ref[...]
    chamfer = jnp.sum(clamped * valid) / jnp.sum(valid)

    loss_b = jnp.where(n_ib < 5.0, fallback, chamfer)
    out_ref[pl.ds(b, 1), :] = loss_b.reshape(1, 1)


@jax.jit
def _run(pred_full, observed, G_T, ref_T, valid):
    Bn = pred_full.shape[0]
    losses = pl.pallas_call(
        _chamfer_kernel,
        grid=(Bn,),
        in_specs=[
            pl.BlockSpec(pred_full.shape, lambda b: (0, 0)),
            pl.BlockSpec(observed.shape, lambda b: (0, 0, 0)),
            pl.BlockSpec(G_T.shape, lambda b: (0, 0)),
            pl.BlockSpec(ref_T.shape, lambda b: (0, 0)),
            pl.BlockSpec(valid.shape, lambda b: (0, 0)),
        ],
        out_specs=pl.BlockSpec((Bn, 1), lambda b: (0, 0)),
        out_shape=jax.ShapeDtypeStruct((Bn, 1), jnp.float32),
    )(pred_full, observed, G_T, ref_T, valid)
    return jnp.mean(losses)


def kernel(pred_coeffs, observed, G, ref, obs_subsample):
    Bn, Dn = pred_coeffs.shape
    M = observed.shape[1]
    pred_full = jnp.zeros((Bn, Dn + 1), jnp.float32).at[:, 1:].set(pred_coeffs)
    G_T = G.T
    ref_T = ref.T
    valid = (jnp.arange(M) < obs_subsample).astype(jnp.float32).reshape(M, 1)
    return _run(pred_full, observed, G_T, ref_T, valid)


# speedup vs baseline: 8.7525x; 1.2416x over previous
import functools

import jax
import jax.numpy as jnp
from jax.experimental import pallas as pl

FOCAL_UM = 6000.0
PITCH_UM = 150.0
SENSOR_W = 9600.0
SENSOR_H = 9600.0
GRID = 64
N_SUB = GRID * GRID
MARGIN = PITCH_UM * 0.5
FAR = 1e9


def _chamfer_kernel(pred_ref, obs_ref, gt_ref, reft_ref, valid_ref, out_ref):
    b = pl.program_id(0)

    coeffs = pred_ref[pl.ds(b, 1), :]
    slopes = jnp.dot(coeffs, gt_ref[...],
                     preferred_element_type=jnp.float32)

    ex = reft_ref[0:1, :] + FOCAL_UM * slopes[:, :N_SUB]
    ey = reft_ref[1:2, :] + FOCAL_UM * slopes[:, N_SUB:]

    in_bounds = ((ex >= -MARGIN) & (ex <= SENSOR_W + MARGIN)
                 & (ey >= -MARGIN) & (ey <= SENSOR_H + MARGIN))
    n_ib = jnp.sum(in_bounds.astype(jnp.float32))

    cx = SENSOR_W / 2.0
    cy = SENSOR_H / 2.0
    center_d = jnp.sqrt((ex - cx) ** 2 + (ey - cy) ** 2)
    fallback = jnp.sum(center_d) * (1.0 / (N_SUB * PITCH_UM)) + 10.0

    ex_eff = jnp.where(in_bounds, ex, FAR)

    exc = ex_eff - cx
    eyc = ey - cy
    e2 = jnp.concatenate([-2.0 * exc, -2.0 * eyc], axis=0)
    n_e = exc * exc + eyc * eyc

    oc = obs_ref[b] - jnp.float32(cx)
    n_o = jnp.sum(oc * oc, axis=1, keepdims=True)
    cross = jnp.dot(oc, e2, preferred_element_type=jnp.float32)

    sq = (cross + n_e) + n_o
    min_sq = jnp.maximum(jnp.min(sq, axis=1, keepdims=True), 0.0)
    min_d = jnp.sqrt(min_sq + 1e-12) * (1.0 / PITCH_UM)
    clamped = jnp.minimum(min_d, 5.0)

    valid = valid_ref[...]
    chamfer = jnp.sum(clamped * valid) / jnp.sum(valid)

    loss_b = jnp.where(n_ib < 5.0, fallback, chamfer)
    out_ref[pl.ds(b, 1), :] = loss_b.reshape(1, 1)


@jax.jit
def _run(pred_full, observed, G_T, ref_T, valid):
    Bn = pred_full.shape[0]
    losses = pl.pallas_call(
        _chamfer_kernel,
        grid=(Bn,),
        in_specs=[
            pl.BlockSpec(pred_full.shape, lambda b: (0, 0)),
            pl.BlockSpec(observed.shape, lambda b: (0, 0, 0)),
            pl.BlockSpec(G_T.shape, lambda b: (0, 0)),
            pl.BlockSpec(ref_T.shape, lambda b: (0, 0)),
            pl.BlockSpec(valid.shape, lambda b: (0, 0)),
        ],
        out_specs=pl.BlockSpec((Bn, 1), lambda b: (0, 0)),
        out_shape=jax.ShapeDtypeStruct((Bn, 1), jnp.float32),
    )(pred_full, observed, G_T, ref_T, valid)
    return jnp.mean(losses)


def kernel(pred_coeffs, observed, G, ref, obs_subsample):
    Bn, Dn = pred_coeffs.shape
    M = observed.shape[1]
    pred_full = jnp.zeros((Bn, Dn + 1), jnp.float32).at[:, 1:].set(pred_coeffs)
    G_T = G.T
    ref_T = ref.T
    valid = (jnp.arange(M) < obs_subsample).astype(jnp.float32).reshape(M, 1)
    return _run(pred_full, observed, G_T, ref_T, valid)
